# baseline (speedup 1.0000x reference)
import functools
import jax
import jax.numpy as jnp
from jax import lax
from jax.experimental import pallas as pl
from jax.experimental.pallas import tpu as pltpu
from jax.experimental.pallas import tpu_sc as plsc

B = 16384
N_ACTIONS = 1000000
DIM = 32
NC = 2
NS = 16
NW = NC * NS
BPW = B // NW

CH = 32768
NFULL = N_ACTIONS // CH
LASTCH = 16896
NALIGN = NFULL * CH + LASTCH
NCHUNK = NFULL + 1
NBUF = 8
TBLK = N_ACTIONS - NALIGN


def _tc_body(tblT_hbm, tail_ref, ctxT_ref, qryT_ref, w_ref,
             t_hbm, tt_ref, d_ref, bufs, stages, isems, osems):
    wc = w_ref[0:1, :]
    wq = w_ref[1:2, :]
    wa = w_ref[2:3, :]
    bias = w_ref[3, 0]
    d_ref[...] = (jnp.dot(wc, ctxT_ref[...],
                          preferred_element_type=jnp.float32)
                  + jnp.dot(wq, qryT_ref[...],
                            preferred_element_type=jnp.float32) + bias)
    tt_ref[...] = jnp.dot(wa, tail_ref[...],
                          preferred_element_type=jnp.float32)

    def in_copy(k, b):
        n = LASTCH if k == NCHUNK - 1 else CH
        return pltpu.make_async_copy(
            tblT_hbm.at[:, pl.ds(k * CH, n)],
            bufs[b].at[:, pl.ds(0, n)], isems[b])

    def out_copy(k, b):
        n = LASTCH if k == NCHUNK - 1 else CH
        return pltpu.make_async_copy(
            stages[b].at[:, pl.ds(0, n)],
            t_hbm.at[:, pl.ds(k * CH, n)], osems[b])

    for b in range(min(NBUF, NCHUNK)):
        in_copy(b, b).start()
    for k in range(NCHUNK):
        b = k % NBUF
        n = LASTCH if k == NCHUNK - 1 else CH
        in_copy(k, b).wait()
        if k >= NBUF:
            out_copy(k - NBUF, b).wait()
        stages[b][:, pl.ds(0, n)] = jnp.dot(
            wa, bufs[b][:, pl.ds(0, n)], preferred_element_type=jnp.float32)
        out_copy(k, b).start()
        if k + NBUF < NCHUNK:
            in_copy(k + NBUF, b).start()
    for k in range(max(0, NCHUNK - NBUF), NCHUNK):
        out_copy(k, k % NBUF).wait()


def _sc_body(t_hbm, tt_hbm, act_hbm, d_hbm, out_hbm,
             idx_v, idm_v, idt_v, tvm_v, tvt_v, dv_v, out_v, sem, sem2):
    wid = lax.axis_index("s") * NC + lax.axis_index("c")
    base = wid * BPW
    pltpu.sync_copy(act_hbm.at[pl.ds(base, BPW)], idx_v)
    for v in range(BPW // 16):
        a = idx_v[pl.ds(v * 16, 16)]
        idm_v[pl.ds(v * 16, 16)] = jnp.minimum(a, NALIGN - 1)
        idt_v[pl.ds(v * 16, 16)] = jnp.clip(a - NALIGN, 0, TBLK - 1)
    g1 = pltpu.async_copy(t_hbm.at[idm_v], tvm_v, sem)
    g2 = pltpu.async_copy(tt_hbm.at[idt_v], tvt_v, sem2)
    pltpu.sync_copy(d_hbm.at[pl.ds(base, BPW)], dv_v)
    g1.wait()
    g2.wait()

    @plsc.parallel_loop(0, BPW // 16, step=1, unroll=4)
    def body(i):
        a = idx_v[pl.ds(i * 16, 16)]
        tv = jnp.where(a < NALIGN, tvm_v[pl.ds(i * 16, 16)],
                       tvt_v[pl.ds(i * 16, 16)])
        out_v[pl.ds(i * 16, 16)] = tv + dv_v[pl.ds(i * 16, 16)]

    pltpu.sync_copy(out_v, out_hbm.at[pl.ds(base, BPW)])


@jax.jit
def _run(ctxT, qryT, action, tblT, tailT, wmat):
    t2, tt2, d2 = pl.pallas_call(
        _tc_body,
        grid=(1,),
        in_specs=[
            pl.BlockSpec(memory_space=pl.ANY),
            pl.BlockSpec((DIM, TBLK), lambda i: (0, 0)),
            pl.BlockSpec((DIM, B), lambda i: (0, 0)),
            pl.BlockSpec((DIM, B), lambda i: (0, 0)),
            pl.BlockSpec((8, DIM), lambda i: (0, 0)),
        ],
        out_specs=[
            pl.BlockSpec(memory_space=pl.ANY),
            pl.BlockSpec((1, TBLK), lambda i: (0, 0)),
            pl.BlockSpec((1, B), lambda i: (0, 0)),
        ],
        out_shape=[
            jax.ShapeDtypeStruct((1, NALIGN), jnp.float32),
            jax.ShapeDtypeStruct((1, TBLK), jnp.float32),
            jax.ShapeDtypeStruct((1, B), jnp.float32),
        ],
        scratch_shapes=[
            [pltpu.VMEM((DIM, CH), jnp.float32) for _ in range(NBUF)],
            [pltpu.VMEM((1, CH), jnp.float32) for _ in range(NBUF)],
            [pltpu.SemaphoreType.DMA for _ in range(NBUF)],
            [pltpu.SemaphoreType.DMA for _ in range(NBUF)],
        ],
        compiler_params=pltpu.CompilerParams(
            vmem_limit_bytes=100 * 1024 * 1024),
    )(tblT, tailT, ctxT, qryT, wmat)

    mesh = plsc.VectorSubcoreMesh(core_axis_name="c", subcore_axis_name="s",
                                  num_cores=NC, num_subcores=NS)
    f = pl.kernel(
        _sc_body,
        out_type=jax.ShapeDtypeStruct((B,), jnp.float32),
        mesh=mesh,
        scratch_types=[
            pltpu.VMEM((BPW,), jnp.int32),
            pltpu.VMEM((BPW,), jnp.int32),
            pltpu.VMEM((BPW,), jnp.int32),
            pltpu.VMEM((BPW,), jnp.float32),
            pltpu.VMEM((BPW,), jnp.float32),
            pltpu.VMEM((BPW,), jnp.float32),
            pltpu.VMEM((BPW,), jnp.float32),
            pltpu.SemaphoreType.DMA,
            pltpu.SemaphoreType.DMA,
        ],
        compiler_params=pltpu.CompilerParams(needs_layout_passes=False,
                                             use_tc_tiling_on_sc=False),
    )
    return f(t2.reshape(NALIGN), tt2.reshape(TBLK), action, d2.reshape(B))


def kernel(context, query, action, action_list, w, b):
    wmat = jnp.zeros((8, DIM), jnp.float32)
    wmat = wmat.at[0].set(w[0:32]).at[1].set(w[32:64]).at[2].set(w[64:96])
    wmat = wmat.at[3, 0].set(b)
    tblT = action_list.T
    tailT = jax.lax.slice(tblT, (0, NALIGN), (DIM, N_ACTIONS))
    return _run(context.T, query.T, action.astype(jnp.int32),
                tblT, tailT, wmat)


# speedup vs baseline: 1.7678x; 1.7678x over previous
import functools
import jax
import jax.numpy as jnp
from jax import lax
from jax.experimental import pallas as pl
from jax.experimental.pallas import tpu as pltpu
from jax.experimental.pallas import tpu_sc as plsc

B = 16384
N_ACTIONS = 1000000
DIM = 32
NC = 2
NS = 16
NW = NC * NS
BPW = B // NW

CBLK = 65536
NSTEP = (N_ACTIONS + CBLK - 1) // CBLK


def _tc_body(tblT_ref, ctxT_ref, qryT_ref, w_ref, t_ref, d_ref):
    wa = w_ref[2:3, :]
    t_ref[...] = jnp.dot(wa, tblT_ref[...],
                         preferred_element_type=jnp.float32)

    @pl.when(pl.program_id(0) == 0)
    def _():
        wc = w_ref[0:1, :]
        wq = w_ref[1:2, :]
        bias = w_ref[3, 0]
        d_ref[...] = (jnp.dot(wc, ctxT_ref[...],
                              preferred_element_type=jnp.float32)
                      + jnp.dot(wq, qryT_ref[...],
                                preferred_element_type=jnp.float32) + bias)


def _sc_body(t_hbm, act_hbm, d_hbm, out_hbm, idx_v, tv_v, dv_v, out_v, sem):
    wid = lax.axis_index("s") * NC + lax.axis_index("c")
    base = wid * BPW
    pltpu.sync_copy(act_hbm.at[pl.ds(base, BPW)], idx_v)
    gather = pltpu.async_copy(t_hbm.at[idx_v], tv_v, sem)
    pltpu.sync_copy(d_hbm.at[pl.ds(base, BPW)], dv_v)
    gather.wait()

    @plsc.parallel_loop(0, BPW // 16, step=1, unroll=8)
    def body(i):
        out_v[pl.ds(i * 16, 16)] = (tv_v[pl.ds(i * 16, 16)]
                                    + dv_v[pl.ds(i * 16, 16)])

    pltpu.sync_copy(out_v, out_hbm.at[pl.ds(base, BPW)])


@jax.jit
def _run(ctxT, qryT, action, tblT, wmat):
    t2, d2 = pl.pallas_call(
        _tc_body,
        grid=(NSTEP,),
        in_specs=[
            pl.BlockSpec((DIM, CBLK), lambda i: (0, i)),
            pl.BlockSpec((DIM, B), lambda i: (0, 0)),
            pl.BlockSpec((DIM, B), lambda i: (0, 0)),
            pl.BlockSpec((8, DIM), lambda i: (0, 0)),
        ],
        out_specs=[
            pl.BlockSpec((1, CBLK), lambda i: (0, i)),
            pl.BlockSpec((1, B), lambda i: (0, 0)),
        ],
        out_shape=[
            jax.ShapeDtypeStruct((1, N_ACTIONS), jnp.float32),
            jax.ShapeDtypeStruct((1, B), jnp.float32),
        ],
    )(tblT, ctxT, qryT, wmat)

    mesh = plsc.VectorSubcoreMesh(core_axis_name="c", subcore_axis_name="s",
                                  num_cores=NC, num_subcores=NS)
    f = pl.kernel(
        _sc_body,
        out_type=jax.ShapeDtypeStruct((B,), jnp.float32),
        mesh=mesh,
        scratch_types=[
            pltpu.VMEM((BPW,), jnp.int32),
            pltpu.VMEM((BPW,), jnp.float32),
            pltpu.VMEM((BPW,), jnp.float32),
            pltpu.VMEM((BPW,), jnp.float32),
            pltpu.SemaphoreType.DMA,
        ],
        compiler_params=pltpu.CompilerParams(needs_layout_passes=False,
                                             use_tc_tiling_on_sc=False),
    )
    return f(t2.reshape(N_ACTIONS), action, d2.reshape(B))


def kernel(context, query, action, action_list, w, b):
    wmat = jnp.zeros((8, DIM), jnp.float32)
    wmat = wmat.at[0].set(w[0:32]).at[1].set(w[32:64]).at[2].set(w[64:96])
    wmat = wmat.at[3, 0].set(b)
    return _run(context.T, query.T, action.astype(jnp.int32),
                action_list.T, wmat)
